# baseline (speedup 1.0000x reference)
import jax
import jax.numpy as jnp
from jax.experimental import pallas as pl
from jax.experimental.pallas import tpu as pltpu

N_NODES = 10000
NFEAT = 128
EMBED = 32
ROWS = 400


def _gcn_kernel(feat_ref, adj_ref, w_ref, out_ref, support_ref):
    i = pl.program_id(0)

    @pl.when(i == 0)
    def _():
        support_ref[...] = jnp.dot(
            feat_ref[...], w_ref[...], preferred_element_type=jnp.float32
        )

    out_ref[pl.ds(i * ROWS, ROWS), :] = jnp.dot(
        adj_ref[...], support_ref[...], preferred_element_type=jnp.float32
    )


@jax.jit
def kernel(features, adj, W):
    grid = (N_NODES // ROWS,)
    return pl.pallas_call(
        _gcn_kernel,
        grid=grid,
        in_specs=[
            pl.BlockSpec((N_NODES, NFEAT), lambda i: (0, 0)),
            pl.BlockSpec((ROWS, N_NODES), lambda i: (i, 0)),
            pl.BlockSpec((NFEAT, EMBED), lambda i: (0, 0)),
        ],
        out_specs=pl.BlockSpec((N_NODES, EMBED), lambda i: (0, 0)),
        out_shape=jax.ShapeDtypeStruct((N_NODES, EMBED), jnp.float32),
        scratch_shapes=[pltpu.VMEM((N_NODES, EMBED), jnp.float32)],
        compiler_params=pltpu.CompilerParams(
            dimension_semantics=("arbitrary",),
        ),
    )(features, adj, W)
